# Initial kernel scaffold; baseline (speedup 1.0000x reference)
import jax
import jax.numpy as jnp
from jax.experimental import pallas as pl


def kernel(feats, part_labels, valid_mask):
    raise NotImplementedError("write your pallas kernel here")


# speedup vs baseline: 190.5989x; 190.5989x over previous
import jax
import jax.numpy as jnp
from jax import lax
from jax.experimental import pallas as pl

_P = 16


def _batch_kernel(feats_ref, lab_ref, vm_ref, out_ref, *, s, k):
    for si in range(s):
        fr = feats_ref[0, :, pl.ds(si * k, k)]
        lab = lab_ref[0, si, :]
        vm = vm_ref[0, si, :]

        pv = lax.broadcasted_iota(jnp.int32, (_P, k), 0)
        oh = (lab[None, :] == pv)
        ohf = oh.astype(jnp.float32)
        voh = ohf * vm[None, :]

        pooled_sum = lax.dot_general(
            fr, voh, (((1,), (1,)), ((), ())),
            preferred_element_type=jnp.float32)
        pooled_count = jnp.sum(voh, axis=1)
        patch_count = jnp.sum(ohf, axis=1)

        cols = []
        for p in range(_P):
            masked = jnp.where(lab[None, :] == p, fr, -100.0)
            cols.append(jnp.max(masked, axis=1))
        pooled_max = jnp.stack(cols, axis=1)
        pooled_max = jnp.where(patch_count[None, :] > 0, pooled_max, 0.0)

        pooled_mean = pooled_sum / jnp.maximum(pooled_count[None, :], 1.0)
        out_ref[0, :, si, :] = pooled_mean + pooled_max


@jax.jit
def kernel(feats, part_labels, valid_mask):
    n, c, s, k = feats.shape
    vmf = valid_mask.astype(jnp.float32)
    feats2 = feats.reshape(n, c, s * k)

    import functools
    body = functools.partial(_batch_kernel, s=s, k=k)

    out = pl.pallas_call(
        body,
        grid=(n,),
        in_specs=[
            pl.BlockSpec((1, c, s * k), lambda i: (i, 0, 0)),
            pl.BlockSpec((1, s, k), lambda i: (i, 0, 0)),
            pl.BlockSpec((1, s, k), lambda i: (i, 0, 0)),
        ],
        out_specs=pl.BlockSpec((1, c, s, _P), lambda i: (i, 0, 0, 0)),
        out_shape=jax.ShapeDtypeStruct((n, c, s, _P), jnp.float32),
    )(feats2, part_labels, vmf)
    return out
